# baseline (speedup 1.0000x reference)
import functools

import jax
import jax.numpy as jnp
from jax import lax
from jax.experimental import pallas as pl
from jax.experimental.pallas import tpu as pltpu
from jax.experimental.pallas import tpu_sc as plsc

_N = 10000
_E = 160000
_D = 128
_L = 4

_NPAD = 10240
_NC, _NS = 2, 16
_NW = _NC * _NS
_CH = 128
_NCHUNK = _E // _CH
_CPW = _NCHUNK // _NW
_EXTRA = _NCHUNK - _CPW * _NW
_MAXC = _CPW + 1
_RPS = _NPAD // _NS
_PIPE = 3
_SPIPE = 2


def _sc_mesh():
    return plsc.VectorSubcoreMesh(
        core_axis_name="c", subcore_axis_name="s", num_cores=_NC, num_subcores=_NS
    )


def _worker_span(wid):
    base = wid * _CPW + jnp.minimum(wid, _EXTRA)
    nch = _CPW + (wid < _EXTRA).astype(jnp.int32)
    return base, nch


def _per_worker_idx(idx):
    flat = jnp.concatenate([idx, jnp.zeros((2 * _CH,), jnp.int32)])
    rows = []
    for w in range(_NW):
        b = w * _CPW + min(w, _EXTRA)
        rows.append(lax.slice(flat, (b * _CH,), (b * _CH + _MAXC * _CH,)))
    return jnp.stack(rows).reshape(_NW, _MAXC, _CH)


def _gather(table, idx_list):
    n = len(idx_list)
    mesh = _sc_mesh()

    @functools.partial(
        pl.kernel,
        out_type=[jax.ShapeDtypeStruct((_E, _D), jnp.float32)] * n,
        mesh=mesh,
        scratch_types=[
            pltpu.VMEM((_MAXC, _CH), jnp.int32),
            pltpu.VMEM((_PIPE, _CH, _D), jnp.float32),
            pltpu.SemaphoreType.DMA((_PIPE,)),
            pltpu.SemaphoreType.DMA((_PIPE,)),
        ],
    )
    def k(table_h, *refs):
        idx_hs = refs[:n]
        out_hs = refs[n : 2 * n]
        idx_v, bufs, gsem, wsem = refs[2 * n :]
        wid = lax.axis_index("s") * _NC + lax.axis_index("c")
        base, nch = _worker_span(wid)

        def run(idx_h, out_h):
            pltpu.sync_copy(idx_h.at[wid], idx_v)

            def gather_desc(c):
                s = lax.rem(c, _PIPE)
                return pltpu.make_async_copy(
                    table_h.at[idx_v.at[c]], bufs.at[s], gsem.at[s]
                )

            def write_desc(c):
                s = lax.rem(c, _PIPE)
                return pltpu.make_async_copy(
                    bufs.at[s], out_h.at[pl.ds((base + c) * _CH, _CH)], wsem.at[s]
                )

            for s in range(_PIPE):
                @pl.when(s < nch)
                def _():
                    gather_desc(s).start()

            def body(c, carry):
                gather_desc(c).wait()
                write_desc(c).start()

                @pl.when(c + _PIPE < nch)
                def _():
                    write_desc(c).wait()
                    gather_desc(c + _PIPE).start()

                return carry

            lax.fori_loop(0, nch, body, 0)

            for s in range(_PIPE):
                c_t = nch - _PIPE + s

                @pl.when(c_t >= 0)
                def _():
                    write_desc(c_t).wait()

        for idx_h, out_h in zip(idx_hs, out_hs):
            run(idx_h, out_h)

    return k(table, *idx_list)


def _scatter_add(msg, idx2d, zeros):
    mesh = _sc_mesh()

    @functools.partial(
        pl.kernel,
        out_type=jax.ShapeDtypeStruct((_NC, _NPAD, _D), jnp.float32),
        mesh=mesh,
        scratch_types=[
            pltpu.VMEM_SHARED((_NPAD, _D), jnp.float32),
            pltpu.VMEM((_MAXC, _CH), jnp.int32),
            pltpu.VMEM((_SPIPE, _CH, _D), jnp.float32),
            pltpu.SemaphoreType.DMA((_SPIPE,)),
        ],
    )
    def k(msg_h, idx_h, z_h, out_h, acc_s, idx_v, bufs, msem):
        cid = lax.axis_index("c")
        sid = lax.axis_index("s")
        wid = sid * _NC + cid
        base, nch = _worker_span(wid)

        pltpu.sync_copy(idx_h.at[wid], idx_v)

        def load_desc(c):
            s = lax.rem(c, _SPIPE)
            return pltpu.make_async_copy(
                msg_h.at[pl.ds((base + c) * _CH, _CH)], bufs.at[s], msem.at[s]
            )

        for s in range(_SPIPE):
            @pl.when(s < nch)
            def _():
                load_desc(s).start()

        pltpu.sync_copy(
            z_h.at[pl.ds(sid * _RPS, _RPS)], acc_s.at[pl.ds(sid * _RPS, _RPS)]
        )
        plsc.subcore_barrier()

        def body(c, carry):
            load_desc(c).wait()
            pltpu.sync_copy(bufs.at[lax.rem(c, _SPIPE)], acc_s.at[idx_v.at[c]], add=True)

            @pl.when(c + _SPIPE < nch)
            def _():
                load_desc(c + _SPIPE).start()

            return carry

        lax.fori_loop(0, nch, body, 0)

        plsc.subcore_barrier()
        pltpu.sync_copy(
            acc_s.at[pl.ds(sid * _RPS, _RPS)],
            out_h.at[cid, pl.ds(sid * _RPS, _RPS)],
        )

    return k(msg, idx2d, zeros)


def _relu_add(a, b):
    be = 4000

    def body(a_r, b_r, o_r):
        o_r[...] = jnp.maximum(a_r[...] + b_r[...], 0.0)

    return pl.pallas_call(
        body,
        grid=(_E // be,),
        in_specs=[pl.BlockSpec((be, _D), lambda i: (i, 0))] * 2,
        out_specs=pl.BlockSpec((be, _D), lambda i: (i, 0)),
        out_shape=jax.ShapeDtypeStruct((_E, _D), jnp.float32),
    )(a, b)


def _node_mlp(nf, parts, eps_l, w1, b1, w2, b2, g, b, relu_out):
    bn = 1000

    def body(nf_r, p_r, eps_r, w1_r, b1_r, w2_r, b2_r, g_r, b_r, out_r):
        h = (1.0 + eps_r[0, 0]) * nf_r[...] + p_r[0] + p_r[1]
        z = jnp.dot(h, w1_r[...], preferred_element_type=jnp.float32) + b1_r[...]
        z = jnp.maximum(z, 0.0)
        z = jnp.dot(z, w2_r[...], preferred_element_type=jnp.float32) + b2_r[...]
        mu = jnp.mean(z, axis=-1, keepdims=True)
        zc = z - mu
        var = jnp.mean(zc * zc, axis=-1, keepdims=True)
        zn = zc * lax.rsqrt(var + 1e-5) * g_r[...] + b_r[...]
        if relu_out:
            zn = jnp.maximum(zn, 0.0)
        out_r[...] = zn + nf_r[...]

    return pl.pallas_call(
        body,
        grid=(_N // bn,),
        in_specs=[
            pl.BlockSpec((bn, _D), lambda i: (i, 0)),
            pl.BlockSpec((_NC, bn, _D), lambda i: (0, i, 0)),
            pl.BlockSpec((1, 1), lambda i: (0, 0)),
            pl.BlockSpec((_D, 2 * _D), lambda i: (0, 0)),
            pl.BlockSpec((1, 2 * _D), lambda i: (0, 0)),
            pl.BlockSpec((2 * _D, _D), lambda i: (0, 0)),
            pl.BlockSpec((1, _D), lambda i: (0, 0)),
            pl.BlockSpec((1, _D), lambda i: (0, 0)),
            pl.BlockSpec((1, _D), lambda i: (0, 0)),
        ],
        out_specs=pl.BlockSpec((bn, _D), lambda i: (i, 0)),
        out_shape=jax.ShapeDtypeStruct((_N, _D), jnp.float32),
    )(nf, parts, eps_l, w1, b1, w2, b2, g, b)


def _edge_update(ni, nj, ef, w1a, w1b, w1c, b1, g, b, w2, b2, emit_msg):
    be = 2000
    n_out = 2 if emit_msg else 1

    def body(ni_r, nj_r, ef_r, w1a_r, w1b_r, w1c_r, b1_r, g_r, b_r, w2_r, b2_r,
             ef_o, *msg_o):
        m = (
            jnp.dot(ni_r[...].astype(jnp.bfloat16), w1a_r[...],
                    preferred_element_type=jnp.float32)
            + jnp.dot(nj_r[...].astype(jnp.bfloat16), w1b_r[...],
                      preferred_element_type=jnp.float32)
            + jnp.dot(ef_r[...].astype(jnp.bfloat16), w1c_r[...],
                      preferred_element_type=jnp.float32)
            + b1_r[...]
        )
        mu = jnp.mean(m, axis=-1, keepdims=True)
        mc = m - mu
        var = jnp.mean(mc * mc, axis=-1, keepdims=True)
        m = mc * lax.rsqrt(var + 1e-5) * g_r[...] + b_r[...]
        m = jnp.maximum(m, 0.0)
        e2 = (
            jnp.dot(m.astype(jnp.bfloat16), w2_r[...],
                    preferred_element_type=jnp.float32)
            + b2_r[...]
            + ef_r[...]
        )
        ef_o[...] = e2
        if emit_msg:
            msg_o[0][...] = jnp.maximum(ni_r[...] + e2, 0.0)

    out_shape = [jax.ShapeDtypeStruct((_E, _D), jnp.float32)] * n_out
    return pl.pallas_call(
        body,
        grid=(_E // be,),
        in_specs=[
            pl.BlockSpec((be, _D), lambda i: (i, 0)),
            pl.BlockSpec((be, _D), lambda i: (i, 0)),
            pl.BlockSpec((be, _D), lambda i: (i, 0)),
            pl.BlockSpec((_D, 3 * _D), lambda i: (0, 0)),
            pl.BlockSpec((_D, 3 * _D), lambda i: (0, 0)),
            pl.BlockSpec((_D, 3 * _D), lambda i: (0, 0)),
            pl.BlockSpec((1, 3 * _D), lambda i: (0, 0)),
            pl.BlockSpec((1, 3 * _D), lambda i: (0, 0)),
            pl.BlockSpec((1, 3 * _D), lambda i: (0, 0)),
            pl.BlockSpec((3 * _D, _D), lambda i: (0, 0)),
            pl.BlockSpec((1, _D), lambda i: (0, 0)),
        ],
        out_specs=[pl.BlockSpec((be, _D), lambda i: (i, 0))] * n_out,
        out_shape=out_shape,
    )(ni, nj, ef, w1a, w1b, w1c, b1, g, b, w2, b2)


def kernel(x, edge_attr, edge_index, params):
    p = params
    src2d = _per_worker_idx(edge_index[0])
    dst2d = _per_worker_idx(edge_index[1])
    zeros = jnp.zeros((_NPAD, _D), jnp.float32)

    (ni0,) = _gather(x, [src2d])
    msg = _relu_add(ni0, edge_attr)

    nf, ef = x, edge_attr
    for l in range(_L):
        parts = _scatter_add(msg, dst2d, zeros)
        nf = _node_mlp(
            nf,
            parts,
            p["eps"][l].reshape(1, 1),
            p["cW1"][l],
            p["cb1"][l].reshape(1, -1),
            p["cW2"][l],
            p["cb2"][l].reshape(1, -1),
            p["ng"][l].reshape(1, -1),
            p["nb"][l].reshape(1, -1),
            relu_out=(l < _L - 1),
        )
        ni, nj = _gather(nf, [src2d, dst2d])
        w1 = p["eW1"][l].astype(jnp.bfloat16)
        outs = _edge_update(
            ni,
            nj,
            ef,
            w1[:_D],
            w1[_D : 2 * _D],
            w1[2 * _D :],
            p["eb1"][l].reshape(1, -1),
            p["eg"][l].reshape(1, -1),
            p["ebln"][l].reshape(1, -1),
            p["eW2"][l].astype(jnp.bfloat16),
            p["eb2"][l].reshape(1, -1),
            emit_msg=(l < _L - 1),
        )
        if l < _L - 1:
            ef, msg = outs
        else:
            (ef,) = outs
    return nf, ef


# speedup vs baseline: 1.0556x; 1.0556x over previous
import functools

import jax
import jax.numpy as jnp
from jax import lax
from jax.experimental import pallas as pl
from jax.experimental.pallas import tpu as pltpu
from jax.experimental.pallas import tpu_sc as plsc

_N = 10000
_E = 160000
_D = 128
_HD = _D // 2
_L = 4

_NPAD = 10240
_NC, _NS = 2, 16
_NW = _NC * _NS
_CH = 128
_NCHUNK = _E // _CH
_CPW = _NCHUNK // _NW
_EXTRA = _NCHUNK - _CPW * _NW
_MAXC = _CPW + 1
_RPS = _NPAD // _NS
_PIPE = 4
_SPIPE = 2


def _pack_bf16_pair(a, b):
    ua = lax.bitcast_convert_type(a, jnp.uint32)
    ub = lax.bitcast_convert_type(b, jnp.uint32)
    ra = ua + (jnp.uint32(0x7FFF) + ((ua >> 16) & jnp.uint32(1)))
    rb = ub + (jnp.uint32(0x7FFF) + ((ub >> 16) & jnp.uint32(1)))
    packed = (ra & jnp.uint32(0xFFFF0000)) | (rb >> 16)
    return lax.bitcast_convert_type(packed, jnp.int32)


def _unpack_bf16_pair(p):
    u = lax.bitcast_convert_type(p, jnp.uint32)
    hi = lax.bitcast_convert_type(u & jnp.uint32(0xFFFF0000), jnp.float32)
    lo = lax.bitcast_convert_type(u << 16, jnp.float32)
    return hi, lo


def _sc_mesh():
    return plsc.VectorSubcoreMesh(
        core_axis_name="c", subcore_axis_name="s", num_cores=_NC, num_subcores=_NS
    )


def _worker_span(wid):
    base = wid * _CPW + jnp.minimum(wid, _EXTRA)
    nch = _CPW + (wid < _EXTRA).astype(jnp.int32)
    return base, nch


def _per_worker_idx(idx):
    flat = jnp.concatenate([idx, jnp.zeros((2 * _CH,), jnp.int32)])
    rows = []
    for w in range(_NW):
        b = w * _CPW + min(w, _EXTRA)
        rows.append(lax.slice(flat, (b * _CH,), (b * _CH + _MAXC * _CH,)))
    return jnp.stack(rows).reshape(_NW, _MAXC, _CH)


def _pack_rows(arr):
    bn = 2000

    def body(a_r, o_r):
        o_r[...] = _pack_bf16_pair(a_r[:, :_HD], a_r[:, _HD:])

    return pl.pallas_call(
        body,
        grid=(_N // bn,),
        in_specs=[pl.BlockSpec((bn, _D), lambda i: (i, 0))],
        out_specs=pl.BlockSpec((bn, _HD), lambda i: (i, 0)),
        out_shape=jax.ShapeDtypeStruct((_N, _HD), jnp.int32),
    )(arr)


def _gather(table, idx_list):
    n = len(idx_list)
    mesh = _sc_mesh()

    @functools.partial(
        pl.kernel,
        out_type=[jax.ShapeDtypeStruct((_E, _D), jnp.float32)] * n,
        mesh=mesh,
        scratch_types=[
            pltpu.VMEM((_MAXC, _CH), jnp.int32),
            pltpu.VMEM((_PIPE, _CH, _D), jnp.float32),
            pltpu.SemaphoreType.DMA((_PIPE,)),
            pltpu.SemaphoreType.DMA((_PIPE,)),
        ],
    )
    def k(table_h, *refs):
        idx_hs = refs[:n]
        out_hs = refs[n : 2 * n]
        idx_v, bufs, gsem, wsem = refs[2 * n :]
        wid = lax.axis_index("s") * _NC + lax.axis_index("c")
        base, nch = _worker_span(wid)

        def run(idx_h, out_h):
            pltpu.sync_copy(idx_h.at[wid], idx_v)

            def gather_desc(c):
                s = lax.rem(c, _PIPE)
                return pltpu.make_async_copy(
                    table_h.at[idx_v.at[c]], bufs.at[s], gsem.at[s]
                )

            def write_desc(c):
                s = lax.rem(c, _PIPE)
                return pltpu.make_async_copy(
                    bufs.at[s], out_h.at[pl.ds((base + c) * _CH, _CH)], wsem.at[s]
                )

            for s in range(_PIPE):
                @pl.when(s < nch)
                def _():
                    gather_desc(s).start()

            def body(c, carry):
                gather_desc(c).wait()
                write_desc(c).start()

                @pl.when(c + _PIPE < nch)
                def _():
                    write_desc(c).wait()
                    gather_desc(c + _PIPE).start()

                return carry

            lax.fori_loop(0, nch, body, 0)

            for s in range(_PIPE):
                c_t = nch - _PIPE + s

                @pl.when(c_t >= 0)
                def _():
                    write_desc(c_t).wait()

        for idx_h, out_h in zip(idx_hs, out_hs):
            run(idx_h, out_h)

    return k(table, *idx_list)


def _scatter_add(msg, idx2d, zeros):
    mesh = _sc_mesh()

    @functools.partial(
        pl.kernel,
        out_type=jax.ShapeDtypeStruct((_NC, _NPAD, _D), jnp.float32),
        mesh=mesh,
        scratch_types=[
            pltpu.VMEM_SHARED((_NPAD, _D), jnp.float32),
            pltpu.VMEM((_MAXC, _CH), jnp.int32),
            pltpu.VMEM((_SPIPE, _CH, _D), jnp.float32),
            pltpu.SemaphoreType.DMA((_SPIPE,)),
        ],
    )
    def k(msg_h, idx_h, z_h, out_h, acc_s, idx_v, bufs, msem):
        cid = lax.axis_index("c")
        sid = lax.axis_index("s")
        wid = sid * _NC + cid
        base, nch = _worker_span(wid)

        pltpu.sync_copy(idx_h.at[wid], idx_v)

        def load_desc(c):
            s = lax.rem(c, _SPIPE)
            return pltpu.make_async_copy(
                msg_h.at[pl.ds((base + c) * _CH, _CH)], bufs.at[s], msem.at[s]
            )

        for s in range(_SPIPE):
            @pl.when(s < nch)
            def _():
                load_desc(s).start()

        pltpu.sync_copy(
            z_h.at[pl.ds(sid * _RPS, _RPS)], acc_s.at[pl.ds(sid * _RPS, _RPS)]
        )
        plsc.subcore_barrier()

        def body(c, carry):
            load_desc(c).wait()
            pltpu.sync_copy(bufs.at[lax.rem(c, _SPIPE)], acc_s.at[idx_v.at[c]], add=True)

            @pl.when(c + _SPIPE < nch)
            def _():
                load_desc(c + _SPIPE).start()

            return carry

        lax.fori_loop(0, nch, body, 0)

        plsc.subcore_barrier()
        pltpu.sync_copy(
            acc_s.at[pl.ds(sid * _RPS, _RPS)],
            out_h.at[cid, pl.ds(sid * _RPS, _RPS)],
        )

    return k(msg, idx2d, zeros)


def _relu_add(a, b):
    be = 4000

    def body(a_r, b_r, o_r):
        o_r[...] = jnp.maximum(a_r[...] + b_r[...], 0.0)

    return pl.pallas_call(
        body,
        grid=(_E // be,),
        in_specs=[
            pl.BlockSpec((be, _D), lambda i: (i, 0)),
            pl.BlockSpec((be, _D), lambda i: (i, 0)),
        ],
        out_specs=pl.BlockSpec((be, _D), lambda i: (i, 0)),
        out_shape=jax.ShapeDtypeStruct((_E, _D), jnp.float32),
    )(a, b)


def _node_mlp(nf, parts, eps_l, w1, b1, w2, b2, g, b, relu_out):
    bn = 1000

    def body(nf_r, p_r, eps_r, w1_r, b1_r, w2_r, b2_r, g_r, b_r, out_r):
        h = (1.0 + eps_r[0, 0]) * nf_r[...] + p_r[0] + p_r[1]
        z = jnp.dot(h, w1_r[...], preferred_element_type=jnp.float32) + b1_r[...]
        z = jnp.maximum(z, 0.0)
        z = jnp.dot(z, w2_r[...], preferred_element_type=jnp.float32) + b2_r[...]
        mu = jnp.mean(z, axis=-1, keepdims=True)
        zc = z - mu
        var = jnp.mean(zc * zc, axis=-1, keepdims=True)
        zn = zc * lax.rsqrt(var + 1e-5) * g_r[...] + b_r[...]
        if relu_out:
            zn = jnp.maximum(zn, 0.0)
        out_r[...] = zn + nf_r[...]

    return pl.pallas_call(
        body,
        grid=(_N // bn,),
        in_specs=[
            pl.BlockSpec((bn, _D), lambda i: (i, 0)),
            pl.BlockSpec((_NC, bn, _D), lambda i: (0, i, 0)),
            pl.BlockSpec((1, 1), lambda i: (0, 0)),
            pl.BlockSpec((_D, 2 * _D), lambda i: (0, 0)),
            pl.BlockSpec((1, 2 * _D), lambda i: (0, 0)),
            pl.BlockSpec((2 * _D, _D), lambda i: (0, 0)),
            pl.BlockSpec((1, _D), lambda i: (0, 0)),
            pl.BlockSpec((1, _D), lambda i: (0, 0)),
            pl.BlockSpec((1, _D), lambda i: (0, 0)),
        ],
        out_specs=pl.BlockSpec((bn, _D), lambda i: (i, 0)),
        out_shape=jax.ShapeDtypeStruct((_N, _D), jnp.float32),
    )(nf, parts, eps_l, w1, b1, w2, b2, g, b)


def _edge_update(ni, nj, ef, w1a, w1b, w1c, b1, g, b, w2, b2, emit_msg):
    be = 4000
    n_out = 2 if emit_msg else 1

    def body(ni_r, nj_r, ef_r, w1a_r, w1b_r, w1c_r, b1_r, g_r, b_r, w2_r, b2_r,
             ef_o, *msg_o):
        m = (
            jnp.dot(ni_r[...], w1a_r[...], preferred_element_type=jnp.float32)
            + jnp.dot(nj_r[...], w1b_r[...], preferred_element_type=jnp.float32)
            + jnp.dot(ef_r[...], w1c_r[...], preferred_element_type=jnp.float32)
            + b1_r[...]
        )
        mu = jnp.mean(m, axis=-1, keepdims=True)
        mc = m - mu
        var = jnp.mean(mc * mc, axis=-1, keepdims=True)
        m = mc * lax.rsqrt(var + 1e-5) * g_r[...] + b_r[...]
        m = jnp.maximum(m, 0.0)
        e2 = (
            jnp.dot(m, w2_r[...], preferred_element_type=jnp.float32)
            + b2_r[...]
            + ef_r[...]
        )
        ef_o[...] = e2
        if emit_msg:
            msg_o[0][...] = jnp.maximum(ni_r[...] + e2, 0.0)

    out_shape = [jax.ShapeDtypeStruct((_E, _D), jnp.float32)] * n_out
    return pl.pallas_call(
        body,
        grid=(_E // be,),
        in_specs=[
            pl.BlockSpec((be, _D), lambda i: (i, 0)),
            pl.BlockSpec((be, _D), lambda i: (i, 0)),
            pl.BlockSpec((be, _D), lambda i: (i, 0)),
            pl.BlockSpec((_D, 3 * _D), lambda i: (0, 0)),
            pl.BlockSpec((_D, 3 * _D), lambda i: (0, 0)),
            pl.BlockSpec((_D, 3 * _D), lambda i: (0, 0)),
            pl.BlockSpec((1, 3 * _D), lambda i: (0, 0)),
            pl.BlockSpec((1, 3 * _D), lambda i: (0, 0)),
            pl.BlockSpec((1, 3 * _D), lambda i: (0, 0)),
            pl.BlockSpec((3 * _D, _D), lambda i: (0, 0)),
            pl.BlockSpec((1, _D), lambda i: (0, 0)),
        ],
        out_specs=[pl.BlockSpec((be, _D), lambda i: (i, 0))] * n_out,
        out_shape=out_shape,
    )(ni, nj, ef, w1a, w1b, w1c, b1, g, b, w2, b2)


def kernel(x, edge_attr, edge_index, params):
    p = params
    src2d = _per_worker_idx(edge_index[0])
    dst2d = _per_worker_idx(edge_index[1])
    zeros = jnp.zeros((_NPAD, _D), jnp.float32)

    (ni0,) = _gather(x, [src2d])
    msg = _relu_add(ni0, edge_attr)

    nf, ef = x, edge_attr
    for l in range(_L):
        parts = _scatter_add(msg, dst2d, zeros)
        nf = _node_mlp(
            nf,
            parts,
            p["eps"][l].reshape(1, 1),
            p["cW1"][l],
            p["cb1"][l].reshape(1, -1),
            p["cW2"][l],
            p["cb2"][l].reshape(1, -1),
            p["ng"][l].reshape(1, -1),
            p["nb"][l].reshape(1, -1),
            relu_out=(l < _L - 1),
        )
        ni, nj = _gather(nf, [src2d, dst2d])
        w1 = p["eW1"][l]
        outs = _edge_update(
            ni,
            nj,
            ef,
            w1[:_D],
            w1[_D : 2 * _D],
            w1[2 * _D :],
            p["eb1"][l].reshape(1, -1),
            p["eg"][l].reshape(1, -1),
            p["ebln"][l].reshape(1, -1),
            p["eW2"][l],
            p["eb2"][l].reshape(1, -1),
            emit_msg=(l < _L - 1),
        )
        if l < _L - 1:
            ef, msg = outs
        else:
            (ef,) = outs
    return nf, ef


# speedup vs baseline: 1.0878x; 1.0305x over previous
import functools

import jax
import jax.numpy as jnp
from jax import lax
from jax.experimental import pallas as pl
from jax.experimental.pallas import tpu as pltpu
from jax.experimental.pallas import tpu_sc as plsc

_N = 10000
_E = 160000
_D = 128
_L = 4
_H = 2
_EH = _E // _H

_NPAD = 10240
_NC, _NS = 2, 16
_NW = _NC * _NS
_CH = 128
_NCHUNK = _EH // _CH
_CPW = _NCHUNK // _NW
_EXTRA = _NCHUNK - _CPW * _NW
_MAXC = _CPW + 1
_RPS = _NPAD // _NS
_PIPE = 4
_SPIPE = 2


def _sc_mesh():
    return plsc.VectorSubcoreMesh(
        core_axis_name="c", subcore_axis_name="s", num_cores=_NC, num_subcores=_NS
    )


def _worker_span(wid):
    base = wid * _CPW + jnp.minimum(wid, _EXTRA)
    nch = _CPW + (wid < _EXTRA).astype(jnp.int32)
    return base, nch


def _per_worker_idx(idx):
    need = (_NW - 1) * _CPW + min(_NW - 1, _EXTRA) + _MAXC - _NCHUNK
    flat = jnp.concatenate([idx, jnp.zeros((need * _CH,), jnp.int32)])
    rows = []
    for w in range(_NW):
        b = w * _CPW + min(w, _EXTRA)
        rows.append(lax.slice(flat, (b * _CH,), (b * _CH + _MAXC * _CH,)))
    return jnp.stack(rows).reshape(_NW, _MAXC, _CH)


def _gather(table, idx_list):
    n = len(idx_list)
    mesh = _sc_mesh()

    @functools.partial(
        pl.kernel,
        out_type=[jax.ShapeDtypeStruct((_EH, _D), jnp.float32)] * n,
        mesh=mesh,
        scratch_types=[
            pltpu.VMEM((_MAXC, _CH), jnp.int32),
            pltpu.VMEM((_PIPE, _CH, _D), jnp.float32),
            pltpu.SemaphoreType.DMA((_PIPE,)),
            pltpu.SemaphoreType.DMA((_PIPE,)),
        ],
    )
    def k(table_h, *refs):
        idx_hs = refs[:n]
        out_hs = refs[n : 2 * n]
        idx_v, bufs, gsem, wsem = refs[2 * n :]
        wid = lax.axis_index("s") * _NC + lax.axis_index("c")
        base, nch = _worker_span(wid)

        def run(idx_h, out_h):
            pltpu.sync_copy(idx_h.at[wid], idx_v)

            def gather_desc(c):
                s = lax.rem(c, _PIPE)
                return pltpu.make_async_copy(
                    table_h.at[idx_v.at[c]], bufs.at[s], gsem.at[s]
                )

            def write_desc(c):
                s = lax.rem(c, _PIPE)
                return pltpu.make_async_copy(
                    bufs.at[s], out_h.at[pl.ds((base + c) * _CH, _CH)], wsem.at[s]
                )

            for s in range(_PIPE):
                @pl.when(s < nch)
                def _():
                    gather_desc(s).start()

            def body(c, carry):
                gather_desc(c).wait()
                write_desc(c).start()

                @pl.when(c + _PIPE < nch)
                def _():
                    write_desc(c).wait()
                    gather_desc(c + _PIPE).start()

                return carry

            lax.fori_loop(0, nch, body, 0)

            for s in range(_PIPE):
                c_t = nch - _PIPE + s

                @pl.when(c_t >= 0)
                def _():
                    write_desc(c_t).wait()

        for idx_h, out_h in zip(idx_hs, out_hs):
            run(idx_h, out_h)

    return k(table, *idx_list)


def _scatter_add(msg, idx2d, zeros):
    mesh = _sc_mesh()

    @functools.partial(
        pl.kernel,
        out_type=jax.ShapeDtypeStruct((_NC, _NPAD, _D), jnp.float32),
        mesh=mesh,
        scratch_types=[
            pltpu.VMEM_SHARED((_NPAD, _D), jnp.float32),
            pltpu.VMEM((_MAXC, _CH), jnp.int32),
            pltpu.VMEM((_SPIPE, _CH, _D), jnp.float32),
            pltpu.SemaphoreType.DMA((_SPIPE,)),
        ],
    )
    def k(msg_h, idx_h, z_h, out_h, acc_s, idx_v, bufs, msem):
        cid = lax.axis_index("c")
        sid = lax.axis_index("s")
        wid = sid * _NC + cid
        base, nch = _worker_span(wid)

        pltpu.sync_copy(idx_h.at[wid], idx_v)

        def load_desc(c):
            s = lax.rem(c, _SPIPE)
            return pltpu.make_async_copy(
                msg_h.at[pl.ds((base + c) * _CH, _CH)], bufs.at[s], msem.at[s]
            )

        for s in range(_SPIPE):
            @pl.when(s < nch)
            def _():
                load_desc(s).start()

        pltpu.sync_copy(
            z_h.at[pl.ds(sid * _RPS, _RPS)], acc_s.at[pl.ds(sid * _RPS, _RPS)]
        )
        plsc.subcore_barrier()

        def body(c, carry):
            load_desc(c).wait()
            pltpu.sync_copy(bufs.at[lax.rem(c, _SPIPE)], acc_s.at[idx_v.at[c]], add=True)

            @pl.when(c + _SPIPE < nch)
            def _():
                load_desc(c + _SPIPE).start()

            return carry

        lax.fori_loop(0, nch, body, 0)

        plsc.subcore_barrier()
        pltpu.sync_copy(
            acc_s.at[pl.ds(sid * _RPS, _RPS)],
            out_h.at[cid, pl.ds(sid * _RPS, _RPS)],
        )

    return k(msg, idx2d, zeros)


def _relu_add(a, b):
    be = 4000

    def body(a_r, b_r, o_r):
        o_r[...] = jnp.maximum(a_r[...] + b_r[...], 0.0)

    return pl.pallas_call(
        body,
        grid=(_EH // be,),
        in_specs=[pl.BlockSpec((be, _D), lambda i: (i, 0))] * 2,
        out_specs=pl.BlockSpec((be, _D), lambda i: (i, 0)),
        out_shape=jax.ShapeDtypeStruct((_EH, _D), jnp.float32),
    )(a, b)


def _node_mlp(nf, parts0, parts1, eps_l, w1, b1, w2, b2, g, b, relu_out):
    bn = 1000

    def body(nf_r, p0_r, p1_r, eps_r, w1_r, b1_r, w2_r, b2_r, g_r, b_r, out_r):
        aggr = p0_r[0] + p0_r[1] + p1_r[0] + p1_r[1]
        h = (1.0 + eps_r[0, 0]) * nf_r[...] + aggr
        z = jnp.dot(h, w1_r[...], preferred_element_type=jnp.float32) + b1_r[...]
        z = jnp.maximum(z, 0.0)
        z = jnp.dot(z, w2_r[...], preferred_element_type=jnp.float32) + b2_r[...]
        mu = jnp.mean(z, axis=-1, keepdims=True)
        zc = z - mu
        var = jnp.mean(zc * zc, axis=-1, keepdims=True)
        zn = zc * lax.rsqrt(var + 1e-5) * g_r[...] + b_r[...]
        if relu_out:
            zn = jnp.maximum(zn, 0.0)
        out_r[...] = zn + nf_r[...]

    return pl.pallas_call(
        body,
        grid=(_N // bn,),
        in_specs=[
            pl.BlockSpec((bn, _D), lambda i: (i, 0)),
            pl.BlockSpec((_NC, bn, _D), lambda i: (0, i, 0)),
            pl.BlockSpec((_NC, bn, _D), lambda i: (0, i, 0)),
            pl.BlockSpec((1, 1), lambda i: (0, 0)),
            pl.BlockSpec((_D, 2 * _D), lambda i: (0, 0)),
            pl.BlockSpec((1, 2 * _D), lambda i: (0, 0)),
            pl.BlockSpec((2 * _D, _D), lambda i: (0, 0)),
            pl.BlockSpec((1, _D), lambda i: (0, 0)),
            pl.BlockSpec((1, _D), lambda i: (0, 0)),
            pl.BlockSpec((1, _D), lambda i: (0, 0)),
        ],
        out_specs=pl.BlockSpec((bn, _D), lambda i: (i, 0)),
        out_shape=jax.ShapeDtypeStruct((_N, _D), jnp.float32),
    )(nf, parts0, parts1, eps_l, w1, b1, w2, b2, g, b)


def _edge_update(ni, nj, ef, w1a, w1b, w1c, b1, g, b, w2, b2, emit_msg):
    be = 4000
    n_out = 2 if emit_msg else 1

    def body(ni_r, nj_r, ef_r, w1a_r, w1b_r, w1c_r, b1_r, g_r, b_r, w2_r, b2_r,
             ef_o, *msg_o):
        m = (
            jnp.dot(ni_r[...], w1a_r[...], preferred_element_type=jnp.float32)
            + jnp.dot(nj_r[...], w1b_r[...], preferred_element_type=jnp.float32)
            + jnp.dot(ef_r[...], w1c_r[...], preferred_element_type=jnp.float32)
            + b1_r[...]
        )
        mu = jnp.mean(m, axis=-1, keepdims=True)
        mc = m - mu
        var = jnp.mean(mc * mc, axis=-1, keepdims=True)
        m = mc * lax.rsqrt(var + 1e-5) * g_r[...] + b_r[...]
        m = jnp.maximum(m, 0.0)
        e2 = (
            jnp.dot(m, w2_r[...], preferred_element_type=jnp.float32)
            + b2_r[...]
            + ef_r[...]
        )
        ef_o[...] = e2
        if emit_msg:
            msg_o[0][...] = jnp.maximum(ni_r[...] + e2, 0.0)

    out_shape = [jax.ShapeDtypeStruct((_EH, _D), jnp.float32)] * n_out
    return pl.pallas_call(
        body,
        grid=(_EH // be,),
        in_specs=[
            pl.BlockSpec((be, _D), lambda i: (i, 0)),
            pl.BlockSpec((be, _D), lambda i: (i, 0)),
            pl.BlockSpec((be, _D), lambda i: (i, 0)),
            pl.BlockSpec((_D, 3 * _D), lambda i: (0, 0)),
            pl.BlockSpec((_D, 3 * _D), lambda i: (0, 0)),
            pl.BlockSpec((_D, 3 * _D), lambda i: (0, 0)),
            pl.BlockSpec((1, 3 * _D), lambda i: (0, 0)),
            pl.BlockSpec((1, 3 * _D), lambda i: (0, 0)),
            pl.BlockSpec((1, 3 * _D), lambda i: (0, 0)),
            pl.BlockSpec((3 * _D, _D), lambda i: (0, 0)),
            pl.BlockSpec((1, _D), lambda i: (0, 0)),
        ],
        out_specs=[pl.BlockSpec((be, _D), lambda i: (i, 0))] * n_out,
        out_shape=out_shape,
    )(ni, nj, ef, w1a, w1b, w1c, b1, g, b, w2, b2)


def kernel(x, edge_attr, edge_index, params):
    p = params
    src_h = [edge_index[0][h * _EH : (h + 1) * _EH] for h in range(_H)]
    dst_h = [edge_index[1][h * _EH : (h + 1) * _EH] for h in range(_H)]
    src2d = [_per_worker_idx(s) for s in src_h]
    dst2d = [_per_worker_idx(d) for d in dst_h]
    zeros = jnp.zeros((_NPAD, _D), jnp.float32)

    ef = [lax.slice(edge_attr, (h * _EH, 0), ((h + 1) * _EH, _D)) for h in range(_H)]
    msg = [None] * _H
    for h in range(_H):
        (ni0,) = _gather(x, [src2d[h]])
        msg[h] = _relu_add(ni0, ef[h])

    nf = x
    for l in range(_L):
        parts = [_scatter_add(msg[h], dst2d[h], zeros) for h in range(_H)]
        nf = _node_mlp(
            nf,
            parts[0],
            parts[1],
            p["eps"][l].reshape(1, 1),
            p["cW1"][l],
            p["cb1"][l].reshape(1, -1),
            p["cW2"][l],
            p["cb2"][l].reshape(1, -1),
            p["ng"][l].reshape(1, -1),
            p["nb"][l].reshape(1, -1),
            relu_out=(l < _L - 1),
        )
        w1 = p["eW1"][l]
        for h in range(_H):
            ni, nj = _gather(nf, [src2d[h], dst2d[h]])
            outs = _edge_update(
                ni,
                nj,
                ef[h],
                w1[:_D],
                w1[_D : 2 * _D],
                w1[2 * _D :],
                p["eb1"][l].reshape(1, -1),
                p["eg"][l].reshape(1, -1),
                p["ebln"][l].reshape(1, -1),
                p["eW2"][l],
                p["eb2"][l].reshape(1, -1),
                emit_msg=(l < _L - 1),
            )
            if l < _L - 1:
                ef[h], msg[h] = outs
            else:
                (ef[h],) = outs
    return nf, jnp.concatenate(ef, axis=0)
